# baseline (speedup 1.0000x reference)
import functools

import jax
import jax.numpy as jnp
from jax import lax
from jax.experimental import pallas as pl
from jax.experimental.pallas import tpu as pltpu
from jax.experimental.pallas import tpu_sc as plsc

N = 10000
D = 128
E = 320000
B = 1024
H = 50
C = 5
V = 100000

NC, NS, L = 2, 16, 16
NW = NC * NS

VP = 100352
SLICE = VP // NS
NPAD = 10240
ZROWS = 32
HP = 64

AGG_N = 10240
ROWS_PT = AGG_N // NS
EW = E // NW
ECH = 128
NPAIR = 39
EREM = EW - NPAIR * 2 * ECH

BPW = B // NW
BG = 8
CPW = (B * C) // NW

_mesh = plsc.VectorSubcoreMesh(core_axis_name="c", subcore_axis_name="s")


def _news_body(x_ref, w_ref, b_ref, o_ref):
    o_ref[...] = jnp.maximum(
        jnp.dot(x_ref[...], w_ref[...], preferred_element_type=jnp.float32)
        + b_ref[...], 0.0)


def _news_encoder(x, W_news, b_news):
    return pl.pallas_call(
        _news_body,
        grid=(25,),
        in_specs=[pl.BlockSpec((400, D), lambda i: (i, 0)),
                  pl.BlockSpec((D, D), lambda i: (0, 0)),
                  pl.BlockSpec((1, D), lambda i: (0, 0))],
        out_specs=pl.BlockSpec((400, D), lambda i: (i, 0)),
        out_shape=jax.ShapeDtypeStruct((N, D), jnp.float32),
    )(x, W_news, b_news.reshape(1, D))


def _agg_body(ne_hbm, src_hbm, dst_hbm, zrows_hbm, out_hbm,
              aggspm, sidx0, sidx1, didx0, didx1, rows0, rows1,
              sidx_t, didx_t, rows_t, gsem0, gsem1):
    cid = lax.axis_index("c")
    sid = lax.axis_index("s")

    pltpu.sync_copy(zrows_hbm, rows0)
    for k in range(5):
        pltpu.sync_copy(rows0, aggspm.at[pl.ds(sid * ROWS_PT + k * 128, 128)])
    plsc.subcore_barrier()

    wid = cid * NS + sid
    base0 = wid * EW

    def body(g, _):
        ba = base0 + 2 * g * ECH
        bb = ba + ECH
        pltpu.sync_copy(src_hbm.at[pl.ds(ba, ECH)], sidx0)
        pltpu.sync_copy(dst_hbm.at[pl.ds(ba, ECH)], didx0)
        ca = pltpu.async_copy(ne_hbm.at[sidx0], rows0, gsem0)
        pltpu.sync_copy(src_hbm.at[pl.ds(bb, ECH)], sidx1)
        pltpu.sync_copy(dst_hbm.at[pl.ds(bb, ECH)], didx1)
        cb = pltpu.async_copy(ne_hbm.at[sidx1], rows1, gsem1)
        ca.wait()
        pltpu.sync_copy(rows0, aggspm.at[didx0], add=True)
        cb.wait()
        pltpu.sync_copy(rows1, aggspm.at[didx1], add=True)
        return 0

    lax.fori_loop(0, NPAIR, body, 0)

    base = base0 + NPAIR * 2 * ECH
    pltpu.sync_copy(src_hbm.at[pl.ds(base, EREM)], sidx_t)
    pltpu.sync_copy(dst_hbm.at[pl.ds(base, EREM)], didx_t)
    pltpu.async_copy(ne_hbm.at[sidx_t], rows_t, gsem0).wait()
    pltpu.sync_copy(rows_t, aggspm.at[didx_t], add=True)

    plsc.subcore_barrier()
    pltpu.sync_copy(aggspm.at[pl.ds(sid * ROWS_PT, ROWS_PT)],
                    out_hbm.at[cid, pl.ds(sid * ROWS_PT, ROWS_PT)])


_agg_kernel = functools.partial(
    pl.kernel,
    out_type=jax.ShapeDtypeStruct((NC, AGG_N, D), jnp.float32),
    mesh=_mesh,
    scratch_types=[
        pltpu.VMEM_SHARED((AGG_N, D), jnp.float32),
        pltpu.VMEM((ECH,), jnp.int32),
        pltpu.VMEM((ECH,), jnp.int32),
        pltpu.VMEM((ECH,), jnp.int32),
        pltpu.VMEM((ECH,), jnp.int32),
        pltpu.VMEM((ECH, D), jnp.float32),
        pltpu.VMEM((ECH, D), jnp.float32),
        pltpu.VMEM((EREM,), jnp.int32),
        pltpu.VMEM((EREM,), jnp.int32),
        pltpu.VMEM((EREM, D), jnp.float32),
        pltpu.SemaphoreType.DMA,
        pltpu.SemaphoreType.DMA,
    ],
)(_agg_body)


def _gnn_body(ne_ref, parts_ref, w_ref, o_ref):
    i = pl.program_id(0)

    @pl.when(i < 25)
    def _():
        agg = parts_ref[0] + parts_ref[1]
        o_ref[...] = ne_ref[...] + jnp.dot(
            agg, w_ref[...], preferred_element_type=jnp.float32)

    @pl.when(i >= 25)
    def _():
        o_ref[...] = jnp.zeros_like(o_ref)


def _gnn(news_emb, parts, W_gnn):
    return pl.pallas_call(
        _gnn_body,
        grid=(26,),
        in_specs=[pl.BlockSpec((400, D), lambda i: (i, 0)),
                  pl.BlockSpec((NC, 400, D), lambda i: (0, i, 0)),
                  pl.BlockSpec((D, D), lambda i: (0, 0))],
        out_specs=pl.BlockSpec((400, D), lambda i: (i, 0)),
        out_shape=jax.ShapeDtypeStruct((N + ZROWS, D), jnp.float32),
    )(news_emb, parts, W_gnn)


def _max_body(n_ref, o_ref):
    o_ref[0, 0] = jnp.max(n_ref[...])


def _max_id(n_id):
    return pl.pallas_call(
        _max_body,
        in_specs=[pl.BlockSpec((8, 1250), lambda: (0, 0))],
        out_specs=pl.BlockSpec(memory_space=pltpu.SMEM),
        out_shape=jax.ShapeDtypeStruct((1, 1), jnp.int32),
    )(n_id.reshape(8, 1250))


ROUNDS_IN = 4


def _lookup_body(nid_hbm, hist_hbm, cand_hbm, hext_hbm, mx_hbm,
                 pooled_hbm, cnt_hbm, crows_hbm,
                 possp, posl, nidv, mxv,
                 histv, qcall, jall, pooledv, cntv, rowsbuf,
                 creqv, cqc, cjv, crowsv, sem):
    cid = lax.axis_index("c")
    sid = lax.axis_index("s")
    wid = cid * NS + sid
    iota16 = lax.iota(jnp.int32, 16)

    base_id = sid * SLICE

    sent = jnp.full((L,), -1, jnp.int32)

    def initc(i, _):
        posl[pl.ds(i * L, L)] = sent
        return 0

    lax.fori_loop(0, SLICE // L, initc, 0)

    pltpu.sync_copy(nid_hbm, nidv)
    pltpu.sync_copy(mx_hbm, mxv)

    def buildc(ch, _):
        ids = nidv[pl.ds(ch * L, L)]
        rel = ids - base_id
        inr = (rel >= 0) & (rel < SLICE)
        relc = jnp.clip(rel, 0, SLICE - 1)
        val = ch * L + iota16
        for _r in range(ROUNDS_IN):
            w = plsc.load_gather(posl, [relc], mask=inr)
            m = inr & (val > w)
            plsc.store_scatter(posl, [relc], val, mask=m)
        return 0

    lax.fori_loop(0, NPAD // L, buildc, 0)

    pltpu.sync_copy(posl, possp.at[pl.ds(base_id, SLICE)])
    plsc.subcore_barrier()

    mxvec = mxv[...]
    mx = mxvec[0]

    pltpu.sync_copy(hist_hbm.at[pl.ds(wid * BPW * HP, BPW * HP)], histv)

    def qcomp(t, _):
        q = histv[pl.ds(t * L, L)]
        qcall[pl.ds(t * L, L)] = jnp.minimum(q, mx)
        return 0

    lax.fori_loop(0, BPW * HP // L, qcomp, 0)

    copies = []
    for kc in range(BPW * HP // ECH):
        copies.append(pltpu.async_copy(
            possp.at[qcall.at[pl.ds(kc * ECH, ECH)]],
            jall.at[pl.ds(kc * ECH, ECH)], sem))
    for cp in copies:
        cp.wait()

    def jeff(t, carry):
        q = histv[pl.ds(t * L, L)]
        j = jall[pl.ds(t * L, L)]
        m = q != 0
        jall[pl.ds(t * L, L)] = jnp.where(
            m & (j >= 0), j, N + ((t * L + iota16) & 31))
        cnt_part = jnp.sum(m.astype(jnp.float32))
        prev = jnp.where(t % (HP // L) == 0, 0.0, carry)
        cur = prev + cnt_part

        @pl.when(t % (HP // L) == (HP // L) - 1)
        def _():
            bi = t // (HP // L)
            plsc.store_scatter(cntv, [iota16 * 0 + bi],
                               cur + jnp.zeros((L,), jnp.float32),
                               mask=iota16 == 0)

        return cur

    lax.fori_loop(0, BPW * HP // L, jeff, 0.0)
    pltpu.sync_copy(cntv, cnt_hbm.at[pl.ds(wid * BPW, BPW)])

    for g in range(BPW // BG):
        copies = []
        for kc in range(BG * HP // ECH):
            base = g * BG * HP + kc * ECH
            copies.append(pltpu.async_copy(
                hext_hbm.at[jall.at[pl.ds(base, ECH)]],
                rowsbuf.at[pl.ds(kc * ECH, ECH)], sem))
        for cp in copies:
            cp.wait()
        for bi in range(BG):
            def sumt(t, accs):
                row = bi * HP + t
                return tuple(accs[k] + rowsbuf[row, pl.ds(k * L, L)]
                             for k in range(D // L))

            accs = tuple(jnp.zeros((L,), jnp.float32) for _ in range(D // L))
            accs = lax.fori_loop(0, HP, sumt, accs)
            for k in range(D // L):
                pooledv[bi, pl.ds(k * L, L)] = accs[k]
        pltpu.sync_copy(pooledv, pooled_hbm.at[pl.ds(wid * BPW + g * BG, BG)])

    pltpu.sync_copy(cand_hbm.at[pl.ds(wid * CPW, CPW)], creqv)

    def cq(t, _):
        q = creqv[pl.ds(t * L, L)]
        cqc[pl.ds(t * L, L)] = jnp.minimum(q, mx)
        return 0

    lax.fori_loop(0, CPW // L, cq, 0)

    copies = []
    for kc in range(CPW // 80):
        copies.append(pltpu.async_copy(
            possp.at[cqc.at[pl.ds(kc * 80, 80)]],
            cjv.at[pl.ds(kc * 80, 80)], sem))
    for cp in copies:
        cp.wait()

    def cjeff(t, _):
        j = cjv[pl.ds(t * L, L)]
        cjv[pl.ds(t * L, L)] = jnp.where(
            j >= 0, j, N + ((t * L + iota16) & 31))
        return 0

    lax.fori_loop(0, CPW // L, cjeff, 0)

    copies = []
    for kc in range(CPW // 80):
        copies.append(pltpu.async_copy(
            hext_hbm.at[cjv.at[pl.ds(kc * 80, 80)]],
            crowsv.at[pl.ds(kc * 80, 80)], sem))
    for cp in copies:
        cp.wait()
    pltpu.sync_copy(crowsv, crows_hbm.at[pl.ds(wid * CPW, CPW)])


_lookup_kernel = functools.partial(
    pl.kernel,
    out_type=(jax.ShapeDtypeStruct((B, D), jnp.float32),
              jax.ShapeDtypeStruct((B,), jnp.float32),
              jax.ShapeDtypeStruct((B * C, D), jnp.float32)),
    mesh=_mesh,
    scratch_types=[
        pltpu.VMEM_SHARED((VP,), jnp.int32),
        pltpu.VMEM((SLICE,), jnp.int32),
        pltpu.VMEM((NPAD,), jnp.int32),
        pltpu.VMEM((16,), jnp.int32),
        pltpu.VMEM((BPW * HP,), jnp.int32),
        pltpu.VMEM((BPW * HP,), jnp.int32),
        pltpu.VMEM((BPW * HP,), jnp.int32),
        pltpu.VMEM((BG, D), jnp.float32),
        pltpu.VMEM((BPW,), jnp.float32),
        pltpu.VMEM((BG * HP, D), jnp.float32),
        pltpu.VMEM((CPW,), jnp.int32),
        pltpu.VMEM((CPW,), jnp.int32),
        pltpu.VMEM((CPW,), jnp.int32),
        pltpu.VMEM((CPW, D), jnp.float32),
        pltpu.SemaphoreType.DMA,
    ],
    compiler_params=pltpu.CompilerParams(needs_layout_passes=False),
)(_lookup_body)


def _score_body(pooled_ref, cnt_ref, w_ref, cand_ref, o_ref):
    denom = jnp.maximum(cnt_ref[...], 1e-6)
    u = jnp.tanh(jnp.dot(pooled_ref[...] / denom, w_ref[...],
                         preferred_element_type=jnp.float32))
    c3 = cand_ref[...].astype(jnp.bfloat16).astype(jnp.float32)
    ub = u.astype(jnp.bfloat16).astype(jnp.float32)
    o_ref[...] = jnp.sum(c3 * ub[:, None, :], axis=-1)


def _score(pooled, cnt, W_user, cand3):
    return pl.pallas_call(
        _score_body,
        grid=(4,),
        in_specs=[pl.BlockSpec((256, D), lambda i: (i, 0)),
                  pl.BlockSpec((256, 1), lambda i: (i, 0)),
                  pl.BlockSpec((D, D), lambda i: (0, 0)),
                  pl.BlockSpec((256, C, D), lambda i: (i, 0, 0))],
        out_specs=pl.BlockSpec((256, C), lambda i: (i, 0)),
        out_shape=jax.ShapeDtypeStruct((B, C), jnp.float32),
    )(pooled, cnt, W_user, cand3)


def kernel(x, W_news, b_news, W_gnn, W_user, n_id, edge_index, history,
           candidates):
    news_emb = _news_encoder(x, W_news, b_news)

    src = edge_index[0]
    dst = edge_index[1]
    zrows = jnp.zeros((ECH, D), jnp.float32)
    parts = _agg_kernel(news_emb, src, dst, zrows)

    h_ext = _gnn(news_emb, parts, W_gnn)

    mx = _max_id(n_id)
    mx16 = jnp.broadcast_to(mx.reshape(1), (16,))

    n_id_ext = jnp.concatenate(
        [n_id, V + (jnp.arange(NPAD - N, dtype=jnp.int32) % ZROWS)])
    hist_p = jnp.concatenate(
        [history, jnp.zeros((B, HP - H), jnp.int32)], axis=1)
    cand_flat = candidates.reshape(B * C)

    pooled, cnt, crows = _lookup_kernel(n_id_ext, hist_p.reshape(B * HP),
                                        cand_flat, h_ext, mx16)

    cand3 = crows.reshape(B, C, D)
    return _score(pooled, cnt.reshape(B, 1), W_user, cand3)


# speedup vs baseline: 1.1588x; 1.1588x over previous
import functools

import jax
import jax.numpy as jnp
from jax import lax
from jax.experimental import pallas as pl
from jax.experimental.pallas import tpu as pltpu
from jax.experimental.pallas import tpu_sc as plsc

N = 10000
D = 128
E = 320000
B = 1024
H = 50
C = 5
V = 100000

NC, NS, L = 2, 16, 16
NW = NC * NS

VP = 100352
SLICE = VP // NS
NPAD = 10240
ZROWS = 32
HP = 64

AGG_N = 10240
ROWS_PT = AGG_N // NS
EW = E // NW
ECH = 128
NPAIR = 39
EREM = EW - NPAIR * 2 * ECH

BPW = B // NW
BG = 8
CPW = (B * C) // NW

_mesh = plsc.VectorSubcoreMesh(core_axis_name="c", subcore_axis_name="s")


def _news_body(x_ref, w_ref, b_ref, o_ref):
    o_ref[...] = jnp.maximum(
        jnp.dot(x_ref[...], w_ref[...], preferred_element_type=jnp.float32)
        + b_ref[...], 0.0)


def _news_encoder(x, W_news, b_news):
    return pl.pallas_call(
        _news_body,
        grid=(25,),
        in_specs=[pl.BlockSpec((400, D), lambda i: (i, 0)),
                  pl.BlockSpec((D, D), lambda i: (0, 0)),
                  pl.BlockSpec((1, D), lambda i: (0, 0))],
        out_specs=pl.BlockSpec((400, D), lambda i: (i, 0)),
        out_shape=jax.ShapeDtypeStruct((N, D), jnp.float32),
    )(x, W_news, b_news.reshape(1, D))


def _agg_body(ne_hbm, src_hbm, dst_hbm, zrows_hbm, out_hbm,
              aggspm, sidx0, sidx1, didx0, didx1, rows0, rows1,
              sidx_t, didx_t, rows_t, gsem0, gsem1):
    cid = lax.axis_index("c")
    sid = lax.axis_index("s")

    pltpu.sync_copy(zrows_hbm, rows0)
    for k in range(5):
        pltpu.sync_copy(rows0, aggspm.at[pl.ds(sid * ROWS_PT + k * 128, 128)])
    plsc.subcore_barrier()

    wid = cid * NS + sid
    base0 = wid * EW

    def body(g, _):
        ba = base0 + 2 * g * ECH
        bb = ba + ECH
        pltpu.sync_copy(src_hbm.at[pl.ds(ba, ECH)], sidx0)
        pltpu.sync_copy(dst_hbm.at[pl.ds(ba, ECH)], didx0)
        ca = pltpu.async_copy(ne_hbm.at[sidx0], rows0, gsem0)
        pltpu.sync_copy(src_hbm.at[pl.ds(bb, ECH)], sidx1)
        pltpu.sync_copy(dst_hbm.at[pl.ds(bb, ECH)], didx1)
        cb = pltpu.async_copy(ne_hbm.at[sidx1], rows1, gsem1)
        ca.wait()
        pltpu.sync_copy(rows0, aggspm.at[didx0], add=True)
        cb.wait()
        pltpu.sync_copy(rows1, aggspm.at[didx1], add=True)
        return 0

    lax.fori_loop(0, NPAIR, body, 0)

    base = base0 + NPAIR * 2 * ECH
    pltpu.sync_copy(src_hbm.at[pl.ds(base, EREM)], sidx_t)
    pltpu.sync_copy(dst_hbm.at[pl.ds(base, EREM)], didx_t)
    pltpu.async_copy(ne_hbm.at[sidx_t], rows_t, gsem0).wait()
    pltpu.sync_copy(rows_t, aggspm.at[didx_t], add=True)

    plsc.subcore_barrier()
    pltpu.sync_copy(aggspm.at[pl.ds(sid * ROWS_PT, ROWS_PT)],
                    out_hbm.at[cid, pl.ds(sid * ROWS_PT, ROWS_PT)])


_agg_kernel = functools.partial(
    pl.kernel,
    out_type=jax.ShapeDtypeStruct((NC, AGG_N, D), jnp.float32),
    mesh=_mesh,
    scratch_types=[
        pltpu.VMEM_SHARED((AGG_N, D), jnp.float32),
        pltpu.VMEM((ECH,), jnp.int32),
        pltpu.VMEM((ECH,), jnp.int32),
        pltpu.VMEM((ECH,), jnp.int32),
        pltpu.VMEM((ECH,), jnp.int32),
        pltpu.VMEM((ECH, D), jnp.float32),
        pltpu.VMEM((ECH, D), jnp.float32),
        pltpu.VMEM((EREM,), jnp.int32),
        pltpu.VMEM((EREM,), jnp.int32),
        pltpu.VMEM((EREM, D), jnp.float32),
        pltpu.SemaphoreType.DMA,
        pltpu.SemaphoreType.DMA,
    ],
)(_agg_body)


def _gnn_body(ne_ref, parts_ref, w_ref, o_ref):
    agg = parts_ref[0] + parts_ref[1]
    o_ref[...] = ne_ref[...] + jnp.dot(
        agg, w_ref[...], preferred_element_type=jnp.float32)


def _gnn(news_emb, parts, W_gnn):
    return pl.pallas_call(
        _gnn_body,
        grid=(25,),
        in_specs=[pl.BlockSpec((400, D), lambda i: (i, 0)),
                  pl.BlockSpec((NC, 400, D), lambda i: (0, i, 0)),
                  pl.BlockSpec((D, D), lambda i: (0, 0))],
        out_specs=pl.BlockSpec((400, D), lambda i: (i, 0)),
        out_shape=jax.ShapeDtypeStruct((N, D), jnp.float32),
    )(news_emb, parts, W_gnn)


def _max_body(n_ref, o_ref):
    o_ref[0, 0] = jnp.max(n_ref[...])


def _max_id(n_id):
    return pl.pallas_call(
        _max_body,
        in_specs=[pl.BlockSpec((8, 1250), lambda: (0, 0))],
        out_specs=pl.BlockSpec(memory_space=pltpu.SMEM),
        out_shape=jax.ShapeDtypeStruct((1, 1), jnp.int32),
    )(n_id.reshape(8, 1250))


ROUNDS_IN = 4


def _lookup_body(nid_hbm, hist_hbm, cand_hbm, hext_hbm, mx_hbm,
                 pooled_hbm, cnt_hbm, crows_hbm,
                 possp, posl, nidv, mxv,
                 histv, qcall, jall, pooledv, cntv, rowsbuf,
                 creqv, cqc, cjv, crowsv, sem):
    cid = lax.axis_index("c")
    sid = lax.axis_index("s")
    wid = cid * NS + sid
    iota16 = lax.iota(jnp.int32, 16)

    base_id = sid * SLICE

    sent = jnp.full((L,), -1, jnp.int32)

    def initc(i, _):
        posl[pl.ds(i * L, L)] = sent
        return 0

    lax.fori_loop(0, SLICE // L, initc, 0)

    pltpu.sync_copy(nid_hbm, nidv)
    pltpu.sync_copy(mx_hbm, mxv)

    def buildc(ch, _):
        ids = nidv[pl.ds(ch * L, L)]
        rel = ids - base_id
        inr = (rel >= 0) & (rel < SLICE)
        relc = jnp.clip(rel, 0, SLICE - 1)
        val = ch * L + iota16
        for _r in range(ROUNDS_IN):
            w = plsc.load_gather(posl, [relc], mask=inr)
            m = inr & (val > w)
            plsc.store_scatter(posl, [relc], val, mask=m)
        return 0

    lax.fori_loop(0, NPAD // L, buildc, 0)

    pltpu.sync_copy(posl, possp.at[pl.ds(base_id, SLICE)])
    plsc.subcore_barrier()

    mxvec = mxv[...]
    mx = mxvec[0]

    pltpu.sync_copy(hist_hbm.at[pl.ds(wid * BPW * HP, BPW * HP)], histv)

    def qcomp(t, _):
        q = histv[pl.ds(t * L, L)]
        qcall[pl.ds(t * L, L)] = jnp.minimum(q, mx)
        return 0

    lax.fori_loop(0, BPW * HP // L, qcomp, 0)

    copies = []
    for kc in range(BPW * HP // ECH):
        copies.append(pltpu.async_copy(
            possp.at[qcall.at[pl.ds(kc * ECH, ECH)]],
            jall.at[pl.ds(kc * ECH, ECH)], sem))
    for cp in copies:
        cp.wait()

    def jeff(t, carry):
        q = histv[pl.ds(t * L, L)]
        j = jall[pl.ds(t * L, L)]
        m = q != 0
        jall[pl.ds(t * L, L)] = jnp.where(
            m & (j >= 0), j, N + ((t * L + iota16) & 31))
        cnt_part = jnp.sum(m.astype(jnp.float32))
        prev = jnp.where(t % (HP // L) == 0, 0.0, carry)
        cur = prev + cnt_part

        @pl.when(t % (HP // L) == (HP // L) - 1)
        def _():
            bi = t // (HP // L)
            plsc.store_scatter(cntv, [iota16 * 0 + bi],
                               cur + jnp.zeros((L,), jnp.float32),
                               mask=iota16 == 0)

        return cur

    lax.fori_loop(0, BPW * HP // L, jeff, 0.0)
    pltpu.sync_copy(cntv, cnt_hbm.at[pl.ds(wid * BPW, BPW)])

    for g in range(BPW // BG):
        copies = []
        for kc in range(BG * HP // ECH):
            base = g * BG * HP + kc * ECH
            copies.append(pltpu.async_copy(
                hext_hbm.at[jall.at[pl.ds(base, ECH)]],
                rowsbuf.at[pl.ds(kc * ECH, ECH)], sem))
        for cp in copies:
            cp.wait()
        for bi in range(BG):
            def sumt(t, accs):
                row = bi * HP + t
                return tuple(accs[k] + rowsbuf[row, pl.ds(k * L, L)]
                             for k in range(D // L))

            accs = tuple(jnp.zeros((L,), jnp.float32) for _ in range(D // L))
            accs = lax.fori_loop(0, HP, sumt, accs)
            for k in range(D // L):
                pooledv[bi, pl.ds(k * L, L)] = accs[k]
        pltpu.sync_copy(pooledv, pooled_hbm.at[pl.ds(wid * BPW + g * BG, BG)])

    pltpu.sync_copy(cand_hbm.at[pl.ds(wid * CPW, CPW)], creqv)

    def cq(t, _):
        q = creqv[pl.ds(t * L, L)]
        cqc[pl.ds(t * L, L)] = jnp.minimum(q, mx)
        return 0

    lax.fori_loop(0, CPW // L, cq, 0)

    copies = []
    for kc in range(CPW // 80):
        copies.append(pltpu.async_copy(
            possp.at[cqc.at[pl.ds(kc * 80, 80)]],
            cjv.at[pl.ds(kc * 80, 80)], sem))
    for cp in copies:
        cp.wait()

    def cjeff(t, _):
        j = cjv[pl.ds(t * L, L)]
        cjv[pl.ds(t * L, L)] = jnp.where(
            j >= 0, j, N + ((t * L + iota16) & 31))
        return 0

    lax.fori_loop(0, CPW // L, cjeff, 0)

    copies = []
    for kc in range(CPW // 80):
        copies.append(pltpu.async_copy(
            hext_hbm.at[cjv.at[pl.ds(kc * 80, 80)]],
            crowsv.at[pl.ds(kc * 80, 80)], sem))
    for cp in copies:
        cp.wait()
    pltpu.sync_copy(crowsv, crows_hbm.at[pl.ds(wid * CPW, CPW)])


_lookup_kernel = functools.partial(
    pl.kernel,
    out_type=(jax.ShapeDtypeStruct((B, D), jnp.float32),
              jax.ShapeDtypeStruct((B,), jnp.float32),
              jax.ShapeDtypeStruct((B * C, D), jnp.float32)),
    mesh=_mesh,
    scratch_types=[
        pltpu.VMEM_SHARED((VP,), jnp.int32),
        pltpu.VMEM((SLICE,), jnp.int32),
        pltpu.VMEM((NPAD,), jnp.int32),
        pltpu.VMEM((16,), jnp.int32),
        pltpu.VMEM((BPW * HP,), jnp.int32),
        pltpu.VMEM((BPW * HP,), jnp.int32),
        pltpu.VMEM((BPW * HP,), jnp.int32),
        pltpu.VMEM((BG, D), jnp.float32),
        pltpu.VMEM((BPW,), jnp.float32),
        pltpu.VMEM((BG * HP, D), jnp.float32),
        pltpu.VMEM((CPW,), jnp.int32),
        pltpu.VMEM((CPW,), jnp.int32),
        pltpu.VMEM((CPW,), jnp.int32),
        pltpu.VMEM((CPW, D), jnp.float32),
        pltpu.SemaphoreType.DMA,
    ],
    compiler_params=pltpu.CompilerParams(needs_layout_passes=False),
)(_lookup_body)


def _score_body(pooled_ref, cnt_ref, w_ref, cand_ref, o_ref):
    denom = jnp.maximum(cnt_ref[...], 1e-6)
    u = jnp.tanh(jnp.dot(pooled_ref[...] / denom, w_ref[...],
                         preferred_element_type=jnp.float32))
    c3 = cand_ref[...].astype(jnp.bfloat16).astype(jnp.float32)
    ub = u.astype(jnp.bfloat16).astype(jnp.float32)
    o_ref[...] = jnp.sum(c3 * ub[:, None, :], axis=-1)


def _score(pooled, cnt, W_user, cand3):
    return pl.pallas_call(
        _score_body,
        grid=(4,),
        in_specs=[pl.BlockSpec((256, D), lambda i: (i, 0)),
                  pl.BlockSpec((256, 1), lambda i: (i, 0)),
                  pl.BlockSpec((D, D), lambda i: (0, 0)),
                  pl.BlockSpec((256, C, D), lambda i: (i, 0, 0))],
        out_specs=pl.BlockSpec((256, C), lambda i: (i, 0)),
        out_shape=jax.ShapeDtypeStruct((B, C), jnp.float32),
    )(pooled, cnt, W_user, cand3)


def kernel(x, W_news, b_news, W_gnn, W_user, n_id, edge_index, history,
           candidates):
    news_emb = _news_encoder(x, W_news, b_news)

    src = edge_index[0]
    dst = edge_index[1]
    zrows = jnp.zeros((ECH, D), jnp.float32)
    parts = _agg_kernel(news_emb, src, dst, zrows)

    h = _gnn(news_emb, parts, W_gnn)
    h_ext = jnp.concatenate([h, jnp.zeros((ZROWS, D), jnp.float32)], axis=0)

    mx = _max_id(n_id)
    mx16 = jnp.broadcast_to(mx.reshape(1), (16,))

    n_id_ext = jnp.concatenate(
        [n_id, V + (jnp.arange(NPAD - N, dtype=jnp.int32) % ZROWS)])
    hist_p = jnp.concatenate(
        [history, jnp.zeros((B, HP - H), jnp.int32)], axis=1)
    cand_flat = candidates.reshape(B * C)

    pooled, cnt, crows = _lookup_kernel(n_id_ext, hist_p.reshape(B * HP),
                                        cand_flat, h_ext, mx16)

    cand3 = crows.reshape(B, C, D)
    return _score(pooled, cnt.reshape(B, 1), W_user, cand3)


# speedup vs baseline: 1.1635x; 1.0041x over previous
import functools

import jax
import jax.numpy as jnp
from jax import lax
from jax.experimental import pallas as pl
from jax.experimental.pallas import tpu as pltpu
from jax.experimental.pallas import tpu_sc as plsc

N = 10000
D = 128
E = 320000
B = 1024
H = 50
C = 5
V = 100000

NC, NS, L = 2, 16, 16
NW = NC * NS

VP = 100352
SLICE = VP // NS
NPAD = 10240
ZROWS = 32
HP = 64

AGG_N = 10240
ROWS_PT = AGG_N // NS
EW = E // NW
ECH = 128
NPAIR = 39
EREM = EW - NPAIR * 2 * ECH

BPW = B // NW
BG = 8
CPW = (B * C) // NW

_mesh = plsc.VectorSubcoreMesh(core_axis_name="c", subcore_axis_name="s")


def _news_body(x_ref, w_ref, b_ref, o_ref):
    o_ref[...] = jnp.maximum(
        jnp.dot(x_ref[...], w_ref[...], preferred_element_type=jnp.float32)
        + b_ref[...], 0.0)


def _news_encoder(x, W_news, b_news):
    return pl.pallas_call(
        _news_body,
        grid=(25,),
        in_specs=[pl.BlockSpec((400, D), lambda i: (i, 0)),
                  pl.BlockSpec((D, D), lambda i: (0, 0)),
                  pl.BlockSpec((1, D), lambda i: (0, 0))],
        out_specs=pl.BlockSpec((400, D), lambda i: (i, 0)),
        out_shape=jax.ShapeDtypeStruct((N, D), jnp.float32),
    )(x, W_news, b_news.reshape(1, D))


def _agg_body(ne_hbm, src_hbm, dst_hbm, zrows_hbm, out_hbm,
              aggspm, sidx0, sidx1, didx0, didx1, rows0, rows1,
              sidx_t, didx_t, rows_t, gsem0, gsem1, ssem):
    cid = lax.axis_index("c")
    sid = lax.axis_index("s")

    pltpu.sync_copy(zrows_hbm, rows0)
    for k in range(5):
        pltpu.sync_copy(rows0, aggspm.at[pl.ds(sid * ROWS_PT + k * 128, 128)])
    plsc.subcore_barrier()

    wid = cid * NS + sid
    base0 = wid * EW

    def body(g, _):
        ba = base0 + 2 * g * ECH
        bb = ba + ECH
        pltpu.sync_copy(src_hbm.at[pl.ds(ba, ECH)], sidx0)
        pltpu.sync_copy(dst_hbm.at[pl.ds(ba, ECH)], didx0)
        ca = pltpu.async_copy(ne_hbm.at[sidx0], rows0, gsem0)
        pltpu.sync_copy(src_hbm.at[pl.ds(bb, ECH)], sidx1)
        pltpu.sync_copy(dst_hbm.at[pl.ds(bb, ECH)], didx1)
        cb = pltpu.async_copy(ne_hbm.at[sidx1], rows1, gsem1)
        ca.wait()
        sa = pltpu.async_copy(rows0, aggspm.at[didx0], ssem, add=True)
        cb.wait()
        sa.wait()
        pltpu.sync_copy(rows1, aggspm.at[didx1], add=True)
        return 0

    lax.fori_loop(0, NPAIR, body, 0)

    base = base0 + NPAIR * 2 * ECH
    pltpu.sync_copy(src_hbm.at[pl.ds(base, EREM)], sidx_t)
    pltpu.sync_copy(dst_hbm.at[pl.ds(base, EREM)], didx_t)
    pltpu.async_copy(ne_hbm.at[sidx_t], rows_t, gsem0).wait()
    pltpu.sync_copy(rows_t, aggspm.at[didx_t], add=True)

    plsc.subcore_barrier()
    pltpu.sync_copy(aggspm.at[pl.ds(sid * ROWS_PT, ROWS_PT)],
                    out_hbm.at[cid, pl.ds(sid * ROWS_PT, ROWS_PT)])


_agg_kernel = functools.partial(
    pl.kernel,
    out_type=jax.ShapeDtypeStruct((NC, AGG_N, D), jnp.float32),
    mesh=_mesh,
    scratch_types=[
        pltpu.VMEM_SHARED((AGG_N, D), jnp.float32),
        pltpu.VMEM((ECH,), jnp.int32),
        pltpu.VMEM((ECH,), jnp.int32),
        pltpu.VMEM((ECH,), jnp.int32),
        pltpu.VMEM((ECH,), jnp.int32),
        pltpu.VMEM((ECH, D), jnp.float32),
        pltpu.VMEM((ECH, D), jnp.float32),
        pltpu.VMEM((EREM,), jnp.int32),
        pltpu.VMEM((EREM,), jnp.int32),
        pltpu.VMEM((EREM, D), jnp.float32),
        pltpu.SemaphoreType.DMA,
        pltpu.SemaphoreType.DMA,
        pltpu.SemaphoreType.DMA,
    ],
)(_agg_body)


def _gnn_body(ne_ref, parts_ref, w_ref, o_ref):
    agg = parts_ref[0] + parts_ref[1]
    o_ref[...] = ne_ref[...] + jnp.dot(
        agg, w_ref[...], preferred_element_type=jnp.float32)


def _gnn(news_emb, parts, W_gnn):
    return pl.pallas_call(
        _gnn_body,
        grid=(25,),
        in_specs=[pl.BlockSpec((400, D), lambda i: (i, 0)),
                  pl.BlockSpec((NC, 400, D), lambda i: (0, i, 0)),
                  pl.BlockSpec((D, D), lambda i: (0, 0))],
        out_specs=pl.BlockSpec((400, D), lambda i: (i, 0)),
        out_shape=jax.ShapeDtypeStruct((N, D), jnp.float32),
    )(news_emb, parts, W_gnn)


def _max_body(n_ref, o_ref):
    o_ref[0, 0] = jnp.max(n_ref[...])


def _max_id(n_id):
    return pl.pallas_call(
        _max_body,
        in_specs=[pl.BlockSpec((8, 1250), lambda: (0, 0))],
        out_specs=pl.BlockSpec(memory_space=pltpu.SMEM),
        out_shape=jax.ShapeDtypeStruct((1, 1), jnp.int32),
    )(n_id.reshape(8, 1250))


ROUNDS_IN = 4


def _lookup_body(nid_hbm, hist_hbm, cand_hbm, hext_hbm, mx_hbm,
                 pooled_hbm, cnt_hbm, crows_hbm,
                 possp, posl, nidv, mxv,
                 histv, qcall, jall, pooledv, cntv, rowsbuf,
                 creqv, cqc, cjv, crowsv, sem):
    cid = lax.axis_index("c")
    sid = lax.axis_index("s")
    wid = cid * NS + sid
    iota16 = lax.iota(jnp.int32, 16)

    base_id = sid * SLICE

    sent = jnp.full((L,), -1, jnp.int32)

    def initc(i, _):
        posl[pl.ds(i * L, L)] = sent
        return 0

    lax.fori_loop(0, SLICE // L, initc, 0)

    pltpu.sync_copy(nid_hbm, nidv)
    pltpu.sync_copy(mx_hbm, mxv)

    def buildc(ch, _):
        ids = nidv[pl.ds(ch * L, L)]
        rel = ids - base_id
        inr = (rel >= 0) & (rel < SLICE)
        relc = jnp.clip(rel, 0, SLICE - 1)
        val = ch * L + iota16
        for _r in range(ROUNDS_IN):
            w = plsc.load_gather(posl, [relc], mask=inr)
            m = inr & (val > w)
            plsc.store_scatter(posl, [relc], val, mask=m)
        return 0

    lax.fori_loop(0, NPAD // L, buildc, 0)

    pltpu.sync_copy(posl, possp.at[pl.ds(base_id, SLICE)])
    plsc.subcore_barrier()

    mxvec = mxv[...]
    mx = mxvec[0]

    pltpu.sync_copy(hist_hbm.at[pl.ds(wid * BPW * HP, BPW * HP)], histv)

    def qcomp(t, _):
        q = histv[pl.ds(t * L, L)]
        qcall[pl.ds(t * L, L)] = jnp.minimum(q, mx)
        return 0

    lax.fori_loop(0, BPW * HP // L, qcomp, 0)

    copies = []
    for kc in range(BPW * HP // ECH):
        copies.append(pltpu.async_copy(
            possp.at[qcall.at[pl.ds(kc * ECH, ECH)]],
            jall.at[pl.ds(kc * ECH, ECH)], sem))
    for cp in copies:
        cp.wait()

    def jeff(t, carry):
        q = histv[pl.ds(t * L, L)]
        j = jall[pl.ds(t * L, L)]
        m = q != 0
        jall[pl.ds(t * L, L)] = jnp.where(
            m & (j >= 0), j, N + ((t * L + iota16) & 31))
        cnt_part = jnp.sum(m.astype(jnp.float32))
        prev = jnp.where(t % (HP // L) == 0, 0.0, carry)
        cur = prev + cnt_part

        @pl.when(t % (HP // L) == (HP // L) - 1)
        def _():
            bi = t // (HP // L)
            plsc.store_scatter(cntv, [iota16 * 0 + bi],
                               cur + jnp.zeros((L,), jnp.float32),
                               mask=iota16 == 0)

        return cur

    lax.fori_loop(0, BPW * HP // L, jeff, 0.0)
    pltpu.sync_copy(cntv, cnt_hbm.at[pl.ds(wid * BPW, BPW)])

    for g in range(BPW // BG):
        copies = []
        for kc in range(BG * HP // ECH):
            base = g * BG * HP + kc * ECH
            copies.append(pltpu.async_copy(
                hext_hbm.at[jall.at[pl.ds(base, ECH)]],
                rowsbuf.at[pl.ds(kc * ECH, ECH)], sem))
        for cp in copies:
            cp.wait()
        for bi in range(BG):
            def sumt(t, accs):
                row = bi * HP + t
                return tuple(accs[k] + rowsbuf[row, pl.ds(k * L, L)]
                             for k in range(D // L))

            accs = tuple(jnp.zeros((L,), jnp.float32) for _ in range(D // L))
            accs = lax.fori_loop(0, HP, sumt, accs)
            for k in range(D // L):
                pooledv[bi, pl.ds(k * L, L)] = accs[k]
        pltpu.sync_copy(pooledv, pooled_hbm.at[pl.ds(wid * BPW + g * BG, BG)])

    pltpu.sync_copy(cand_hbm.at[pl.ds(wid * CPW, CPW)], creqv)

    def cq(t, _):
        q = creqv[pl.ds(t * L, L)]
        cqc[pl.ds(t * L, L)] = jnp.minimum(q, mx)
        return 0

    lax.fori_loop(0, CPW // L, cq, 0)

    copies = []
    for kc in range(CPW // 80):
        copies.append(pltpu.async_copy(
            possp.at[cqc.at[pl.ds(kc * 80, 80)]],
            cjv.at[pl.ds(kc * 80, 80)], sem))
    for cp in copies:
        cp.wait()

    def cjeff(t, _):
        j = cjv[pl.ds(t * L, L)]
        cjv[pl.ds(t * L, L)] = jnp.where(
            j >= 0, j, N + ((t * L + iota16) & 31))
        return 0

    lax.fori_loop(0, CPW // L, cjeff, 0)

    copies = []
    for kc in range(CPW // 80):
        copies.append(pltpu.async_copy(
            hext_hbm.at[cjv.at[pl.ds(kc * 80, 80)]],
            crowsv.at[pl.ds(kc * 80, 80)], sem))
    for cp in copies:
        cp.wait()
    pltpu.sync_copy(crowsv, crows_hbm.at[pl.ds(wid * CPW, CPW)])


_lookup_kernel = functools.partial(
    pl.kernel,
    out_type=(jax.ShapeDtypeStruct((B, D), jnp.float32),
              jax.ShapeDtypeStruct((B,), jnp.float32),
              jax.ShapeDtypeStruct((B * C, D), jnp.float32)),
    mesh=_mesh,
    scratch_types=[
        pltpu.VMEM_SHARED((VP,), jnp.int32),
        pltpu.VMEM((SLICE,), jnp.int32),
        pltpu.VMEM((NPAD,), jnp.int32),
        pltpu.VMEM((16,), jnp.int32),
        pltpu.VMEM((BPW * HP,), jnp.int32),
        pltpu.VMEM((BPW * HP,), jnp.int32),
        pltpu.VMEM((BPW * HP,), jnp.int32),
        pltpu.VMEM((BG, D), jnp.float32),
        pltpu.VMEM((BPW,), jnp.float32),
        pltpu.VMEM((BG * HP, D), jnp.float32),
        pltpu.VMEM((CPW,), jnp.int32),
        pltpu.VMEM((CPW,), jnp.int32),
        pltpu.VMEM((CPW,), jnp.int32),
        pltpu.VMEM((CPW, D), jnp.float32),
        pltpu.SemaphoreType.DMA,
    ],
    compiler_params=pltpu.CompilerParams(needs_layout_passes=False),
)(_lookup_body)


def _score_body(pooled_ref, cnt_ref, w_ref, cand_ref, o_ref):
    denom = jnp.maximum(cnt_ref[...], 1e-6)
    u = jnp.tanh(jnp.dot(pooled_ref[...] / denom, w_ref[...],
                         preferred_element_type=jnp.float32))
    c3 = cand_ref[...].astype(jnp.bfloat16).astype(jnp.float32)
    ub = u.astype(jnp.bfloat16).astype(jnp.float32)
    o_ref[...] = jnp.sum(c3 * ub[:, None, :], axis=-1)


def _score(pooled, cnt, W_user, cand3):
    return pl.pallas_call(
        _score_body,
        grid=(4,),
        in_specs=[pl.BlockSpec((256, D), lambda i: (i, 0)),
                  pl.BlockSpec((256, 1), lambda i: (i, 0)),
                  pl.BlockSpec((D, D), lambda i: (0, 0)),
                  pl.BlockSpec((256, C, D), lambda i: (i, 0, 0))],
        out_specs=pl.BlockSpec((256, C), lambda i: (i, 0)),
        out_shape=jax.ShapeDtypeStruct((B, C), jnp.float32),
    )(pooled, cnt, W_user, cand3)


def kernel(x, W_news, b_news, W_gnn, W_user, n_id, edge_index, history,
           candidates):
    news_emb = _news_encoder(x, W_news, b_news)

    src = edge_index[0]
    dst = edge_index[1]
    zrows = jnp.zeros((ECH, D), jnp.float32)
    parts = _agg_kernel(news_emb, src, dst, zrows)

    h = _gnn(news_emb, parts, W_gnn)
    h_ext = jnp.concatenate([h, jnp.zeros((ZROWS, D), jnp.float32)], axis=0)

    mx = _max_id(n_id)
    mx16 = jnp.broadcast_to(mx.reshape(1), (16,))

    n_id_ext = jnp.concatenate(
        [n_id, V + (jnp.arange(NPAD - N, dtype=jnp.int32) % ZROWS)])
    hist_p = jnp.concatenate(
        [history, jnp.zeros((B, HP - H), jnp.int32)], axis=1)
    cand_flat = candidates.reshape(B * C)

    pooled, cnt, crows = _lookup_kernel(n_id_ext, hist_p.reshape(B * HP),
                                        cand_flat, h_ext, mx16)

    cand3 = crows.reshape(B, C, D)
    return _score(pooled, cnt.reshape(B, 1), W_user, cand3)
